# baseline (speedup 1.0000x reference)
import jax
import jax.numpy as jnp
from jax import lax
from jax.experimental import pallas as pl
from jax.experimental.pallas import tpu as pltpu
from jax.experimental.pallas import tpu_sc as plsc

N = 10000
E = 320000
G = 256
D = 128
H = 3 * D

NSPLIT = 2
EH = E // NSPLIT

NC = 2
NS = 16
NW = NC * NS

RPT = 624
RTAIL = N - NS * RPT

BE = 1280
BN = 1000


def _sc_mesh():
    return plsc.VectorSubcoreMesh(core_axis_name="c", subcore_axis_name="s")


def _pick_chunk(per_worker, limit):
    for c in (200, 160, 80, 40, 16, 8):
        if c <= limit and per_worker % c == 0 and (per_worker // c) % 2 == 1:
            return c
    raise ValueError(per_worker)


def _sc_gather(atom, n2g, src, dst, base, esize):
    ew = esize // NW
    cg = _pick_chunk(ew, 80)
    nchunk = ew // cg

    def body(atom_hbm, n2g_hbm, src_hbm, dst_hbm, hsum_hbm, gsrc_hbm,
             idx_s, idx_d, n2g_v, g_all, rs0, rd0, rs1, rd1, hs0, hs1,
             semg0, semg1, semw0, semw1):
        cid = lax.axis_index("c")
        sid = lax.axis_index("s")
        wid = sid * NC + cid
        gbase = base + wid * ew
        obase = wid * ew

        pltpu.sync_copy(src_hbm.at[pl.ds(gbase, ew)], idx_s)
        pltpu.sync_copy(dst_hbm.at[pl.ds(gbase, ew)], idx_d)
        pltpu.sync_copy(n2g_hbm, n2g_v)

        def fire(k, rs, rd, sem):
            pltpu.async_copy(atom_hbm.at[idx_s.at[pl.ds(k * cg, cg)]], rs, sem)
            pltpu.async_copy(atom_hbm.at[idx_d.at[pl.ds(k * cg, cg)]], rd, sem)

        def drain_g(rs, rd, sem):
            pltpu.make_async_copy(atom_hbm.at[pl.ds(0, cg)], rs, sem).wait()
            pltpu.make_async_copy(atom_hbm.at[pl.ds(0, cg)], rd, sem).wait()

        def drain_w(hs, sem):
            pltpu.make_async_copy(hs, hsum_hbm.at[pl.ds(0, cg)], sem).wait()

        def process(k, rs, rd, hs, semw):
            def row(r, c):
                def col(q, c2):
                    sl = pl.ds(q * 16, 16)
                    hs[r, sl] = rs[r, sl] + rd[r, sl]
                    return c2
                return lax.fori_loop(0, D // 16, col, c)
            lax.fori_loop(0, cg, row, 0)
            pltpu.async_copy(hs, hsum_hbm.at[pl.ds(obase + k * cg, cg)], semw)

        fire(0, rs0, rd0, semg0)

        def gs(j, c):
            sl = pl.ds(j * 16, 16)
            g_all[sl] = plsc.load_gather(n2g_v, [idx_s[sl]])
            return c
        lax.fori_loop(0, ew // 16, gs, 0)
        if ew % 16:
            sl = pl.ds(ew - 16, 16)
            g_all[sl] = plsc.load_gather(n2g_v, [idx_s[sl]])
        pltpu.sync_copy(g_all, gsrc_hbm.at[pl.ds(obase, ew)])

        def pair(t, carry):
            a = 2 * t
            b = a + 1
            fire(b, rs1, rd1, semg1)
            drain_g(rs0, rd0, semg0)

            @pl.when(t > 0)
            def _():
                drain_w(hs0, semw0)
            process(a, rs0, rd0, hs0, semw0)
            fire(b + 1, rs0, rd0, semg0)
            drain_g(rs1, rd1, semg1)

            @pl.when(t > 0)
            def _():
                drain_w(hs1, semw1)
            process(b, rs1, rd1, hs1, semw1)
            return carry

        lax.fori_loop(0, (nchunk - 1) // 2, pair, 0)
        drain_g(rs0, rd0, semg0)
        drain_w(hs0, semw0)
        process(nchunk - 1, rs0, rd0, hs0, semw0)
        drain_w(hs0, semw0)
        drain_w(hs1, semw1)

    f = pl.kernel(
        body,
        out_type=(
            jax.ShapeDtypeStruct((esize, D), jnp.float32),
            jax.ShapeDtypeStruct((esize,), jnp.int32),
        ),
        mesh=_sc_mesh(),
        scratch_types=[
            pltpu.VMEM((ew,), jnp.int32),
            pltpu.VMEM((ew,), jnp.int32),
            pltpu.VMEM((N,), jnp.int32),
            pltpu.VMEM((ew,), jnp.int32),
            pltpu.VMEM((cg, D), jnp.float32),
            pltpu.VMEM((cg, D), jnp.float32),
            pltpu.VMEM((cg, D), jnp.float32),
            pltpu.VMEM((cg, D), jnp.float32),
            pltpu.VMEM((cg, D), jnp.float32),
            pltpu.VMEM((cg, D), jnp.float32),
            pltpu.SemaphoreType.DMA,
            pltpu.SemaphoreType.DMA,
            pltpu.SemaphoreType.DMA,
            pltpu.SemaphoreType.DMA,
        ],
        compiler_params=pltpu.CompilerParams(needs_layout_passes=False),
    )
    return f(atom, n2g, src, dst)


def _sc_scatter(data, src, dst, zeros_nd, base_idx, esize, gather_rows):
    ewl = esize // NW
    cr = 40
    nchunk = ewl // cr

    def body(data_hbm, src_hbm, dst_hbm, zeros_hbm, out_hbm,
             idx_s, i0, i1, i2, i3, r0, r1, r2, r3, accum,
             semx0, semx1, semx2, semx3,
             semr0, semr1, semr2, semr3,
             semc0, semc1, semc2, semc3):
        idd = (i0, i1, i2, i3)
        rows = (r0, r1, r2, r3)
        semx = (semx0, semx1, semx2, semx3)
        semr = (semr0, semr1, semr2, semr3)
        semc = (semc0, semc1, semc2, semc3)
        cid = lax.axis_index("c")
        sid = lax.axis_index("s")
        wid = sid * NC + cid
        ibase = base_idx + wid * ewl
        dbase = wid * ewl

        rr0 = sid * RPT
        pltpu.sync_copy(zeros_hbm.at[pl.ds(rr0, RPT)],
                        accum.at[pl.ds(rr0, RPT)])

        @pl.when(sid == 0)
        def _():
            pltpu.sync_copy(zeros_hbm.at[pl.ds(NS * RPT, RTAIL)],
                            accum.at[pl.ds(NS * RPT, RTAIL)])
        if gather_rows:
            pltpu.sync_copy(src_hbm.at[pl.ds(ibase, ewl)], idx_s)
        plsc.subcore_barrier()

        def fire_idx(k, slot):
            pltpu.async_copy(dst_hbm.at[pl.ds(ibase + k * cr, cr)],
                             idd[slot], semx[slot])

        def drain_idx(slot):
            pltpu.make_async_copy(dst_hbm.at[pl.ds(0, cr)], idd[slot],
                                  semx[slot]).wait()

        def fire_rows(k, slot):
            if gather_rows:
                pltpu.async_copy(data_hbm.at[idx_s.at[pl.ds(k * cr, cr)]],
                                 rows[slot], semr[slot])
            else:
                pltpu.async_copy(data_hbm.at[pl.ds(dbase + k * cr, cr)],
                                 rows[slot], semr[slot])

        def drain_rows(slot):
            pltpu.make_async_copy(data_hbm.at[pl.ds(0, cr)], rows[slot],
                                  semr[slot]).wait()

        def fire_sc(k, slot):
            pltpu.async_copy(rows[slot], accum.at[idd[slot]], semc[slot],
                             add=True)

        def drain_sc(slot):
            pltpu.make_async_copy(rows[slot], accum.at[pl.ds(0, cr)],
                                  semc[slot]).wait()

        for j in range(3):
            fire_idx(j, j)
            fire_rows(j, j)

        def quad(t, c):
            for j in range(4):
                k = 4 * t + j

                @pl.when(k < nchunk)
                def _():
                    drain_rows(j)
                    drain_idx(j)
                    fire_sc(k, j)
                    kk = k + 3

                    @pl.when(kk < nchunk)
                    def _():
                        jj = (j + 3) % 4

                        @pl.when(kk >= 4)
                        def _():
                            drain_sc(jj)
                        fire_idx(kk, jj)
                        fire_rows(kk, jj)
            return c

        lax.fori_loop(0, (nchunk + 3) // 4, quad, 0)
        for j in range(4):
            drain_sc(j)

        plsc.subcore_barrier()
        pltpu.sync_copy(accum.at[pl.ds(rr0, RPT)],
                        out_hbm.at[cid, pl.ds(rr0, RPT)])

        @pl.when(sid == 0)
        def _():
            pltpu.sync_copy(accum.at[pl.ds(NS * RPT, RTAIL)],
                            out_hbm.at[cid, pl.ds(NS * RPT, RTAIL)])

    f = pl.kernel(
        body,
        out_type=jax.ShapeDtypeStruct((NC, N, D), jnp.float32),
        mesh=_sc_mesh(),
        scratch_types=(
            [pltpu.VMEM((ewl,), jnp.int32)]
            + [pltpu.VMEM((cr,), jnp.int32) for _ in range(4)]
            + [pltpu.VMEM((cr, D), jnp.float32) for _ in range(4)]
            + [pltpu.VMEM_SHARED((N, D), jnp.float32)]
            + [pltpu.SemaphoreType.DMA for _ in range(12)]
        ),
    )
    return f(data, src, dst, zeros_nd)


def _tc_edge_body(hsum_ref, bond_ref, gsrc_ref, u_ref, wb1_ref,
                  bb1_ref, wb2_ref, bb2_ref, out_ref, ugb_ref):
    i = pl.program_id(0)

    @pl.when(i == 0)
    def _():
        ugb_ref[...] = (
            jnp.dot(u_ref[...].astype(jnp.bfloat16), wb1_ref[2 * D:, :],
                    preferred_element_type=jnp.float32)
            + bb1_ref[...]).astype(jnp.bfloat16)

    g = gsrc_ref[0]
    ohT = (lax.broadcasted_iota(jnp.int32, (G, BE), 0) == g
           ).astype(jnp.bfloat16)
    xu = lax.dot_general(ohT, ugb_ref[...], (((0,), (0,)), ((), ())),
                         preferred_element_type=jnp.float32)
    sh = hsum_ref[...].astype(jnp.bfloat16)
    shb = jnp.concatenate([sh, bond_ref[...].astype(jnp.bfloat16)], axis=1)
    x = (jnp.dot(shb, wb1_ref[:2 * D, :], preferred_element_type=jnp.float32)
         + xu)
    x = jnp.maximum(x, 0.0).astype(jnp.bfloat16)
    out_ref[...] = (jnp.dot(x, wb2_ref[...],
                            preferred_element_type=jnp.float32)
                    + bb2_ref[...])


def _tc_edge(hsum, bond, gsrc3, u, Wb1, bb1, Wb2, bb2, boff):
    nsteps = EH // BE
    return pl.pallas_call(
        _tc_edge_body,
        grid=(nsteps,),
        in_specs=[
            pl.BlockSpec((BE, D), lambda i: (i, 0)),
            pl.BlockSpec((BE, D), lambda i: (i + boff, 0)),
            pl.BlockSpec((1, 1, BE), lambda i: (i, 0, 0)),
            pl.BlockSpec((G, D), lambda i: (0, 0)),
            pl.BlockSpec((H, H), lambda i: (0, 0)),
            pl.BlockSpec((1, H), lambda i: (0, 0)),
            pl.BlockSpec((H, D), lambda i: (0, 0)),
            pl.BlockSpec((1, D), lambda i: (0, 0)),
        ],
        out_specs=pl.BlockSpec((BE, D), lambda i: (i, 0)),
        out_shape=jax.ShapeDtypeStruct((EH, D), jnp.float32),
        scratch_shapes=[pltpu.VMEM((G, H), jnp.bfloat16)],
        compiler_params=pltpu.CompilerParams(
            dimension_semantics=("arbitrary",)),
    )(hsum, bond, gsrc3, u, Wb1, bb1, Wb2, bb2)


def _tc_final_body(atom_ref, sumh_ref, sume1_ref, sume2_ref, n2g_ref, u_ref,
                   wa1_ref, ba1_ref, wa2_ref, ba2_ref,
                   wg1_ref, bg1_ref, wg2_ref, bg2_ref,
                   hnew_ref, unew_ref,
                   segh_ref, sege_ref, uga_ref):
    i = pl.program_id(0)
    nsteps = pl.num_programs(0)

    @pl.when(i == 0)
    def _():
        segh_ref[...] = jnp.zeros((G, D), jnp.float32)
        sege_ref[...] = jnp.zeros((G, D), jnp.float32)
        uga_ref[...] = (
            jnp.dot(u_ref[...], wa1_ref[2 * D:, :],
                    preferred_element_type=jnp.float32) + ba1_ref[...])

    sum_h = sumh_ref[0] + sumh_ref[1] + atom_ref[...]
    sum_e = (sume1_ref[0] + sume1_ref[1]) + (sume2_ref[0] + sume2_ref[1])
    g = n2g_ref[0]
    ohT = (lax.broadcasted_iota(jnp.int32, (G, BN), 0) == g
           ).astype(jnp.float32)
    xu = lax.dot_general(ohT, uga_ref[...], (((0,), (0,)), ((), ())),
                         preferred_element_type=jnp.float32)
    x = (jnp.dot(sum_h, wa1_ref[:D, :], preferred_element_type=jnp.float32)
         + jnp.dot(sum_e, wa1_ref[D:2 * D, :],
                   preferred_element_type=jnp.float32)
         + xu)
    x = jnp.maximum(x, 0.0)
    hn = (jnp.dot(x, wa2_ref[...], preferred_element_type=jnp.float32)
          + ba2_ref[...])
    hnew_ref[...] = hn
    segh_ref[...] += jnp.dot(ohT, hn, preferred_element_type=jnp.float32)
    sege_ref[...] += jnp.dot(ohT, sum_e, preferred_element_type=jnp.float32)

    @pl.when(i == nsteps - 1)
    def _():
        xg = (jnp.dot(segh_ref[...], wg1_ref[:D, :],
                      preferred_element_type=jnp.float32)
              + jnp.dot(0.5 * sege_ref[...], wg1_ref[D:2 * D, :],
                        preferred_element_type=jnp.float32)
              + jnp.dot(u_ref[...], wg1_ref[2 * D:, :],
                        preferred_element_type=jnp.float32)
              + bg1_ref[...])
        xg = jnp.maximum(xg, 0.0)
        unew_ref[...] = (jnp.dot(xg, wg2_ref[...],
                                 preferred_element_type=jnp.float32)
                         + bg2_ref[...])


def _tc_final(atom, sumh_parts, sume1_parts, sume2_parts, n2g3, u,
              Wa1, ba1, Wa2, ba2, Wg1, bg1, Wg2, bg2):
    nsteps = N // BN
    return pl.pallas_call(
        _tc_final_body,
        grid=(nsteps,),
        in_specs=[
            pl.BlockSpec((BN, D), lambda i: (i, 0)),
            pl.BlockSpec((NC, BN, D), lambda i: (0, i, 0)),
            pl.BlockSpec((NC, BN, D), lambda i: (0, i, 0)),
            pl.BlockSpec((NC, BN, D), lambda i: (0, i, 0)),
            pl.BlockSpec((1, 1, BN), lambda i: (i, 0, 0)),
            pl.BlockSpec((G, D), lambda i: (0, 0)),
            pl.BlockSpec((H, H), lambda i: (0, 0)),
            pl.BlockSpec((1, H), lambda i: (0, 0)),
            pl.BlockSpec((H, D), lambda i: (0, 0)),
            pl.BlockSpec((1, D), lambda i: (0, 0)),
            pl.BlockSpec((H, H), lambda i: (0, 0)),
            pl.BlockSpec((1, H), lambda i: (0, 0)),
            pl.BlockSpec((H, D), lambda i: (0, 0)),
            pl.BlockSpec((1, D), lambda i: (0, 0)),
        ],
        out_specs=[
            pl.BlockSpec((BN, D), lambda i: (i, 0)),
            pl.BlockSpec((G, D), lambda i: (0, 0)),
        ],
        out_shape=[
            jax.ShapeDtypeStruct((N, D), jnp.float32),
            jax.ShapeDtypeStruct((G, D), jnp.float32),
        ],
        scratch_shapes=[
            pltpu.VMEM((G, D), jnp.float32),
            pltpu.VMEM((G, D), jnp.float32),
            pltpu.VMEM((G, H), jnp.float32),
        ],
        compiler_params=pltpu.CompilerParams(
            dimension_semantics=("arbitrary",)),
    )(atom, sumh_parts, sume1_parts, sume2_parts, n2g3, u,
      Wa1, ba1, Wa2, ba2, Wg1, bg1, Wg2, bg2)


def kernel(atom, bond, global_feats, edge_index, node2graph,
           Wb1, bb1, Wb2, bb2, Wa1, ba1, Wa2, ba2, Wg1, bg1, Wg2, bg2):
    src = edge_index[0].astype(jnp.int32)
    dst = edge_index[1].astype(jnp.int32)
    n2g = node2graph.astype(jnp.int32)
    zeros_nd = jnp.zeros((N, D), jnp.float32)
    wb1_bf = Wb1.astype(jnp.bfloat16)
    wb2_bf = Wb2.astype(jnp.bfloat16)

    hsum1, gsrc1 = _sc_gather(atom, n2g, src, dst, 0, EH)
    hsum2, gsrc2 = _sc_gather(atom, n2g, src, dst, EH, EH)
    sumh_parts = _sc_scatter(atom, src, dst, zeros_nd, 0, E,
                             gather_rows=True)
    e_new1 = _tc_edge(hsum1, bond, gsrc1.reshape(EH // BE, 1, BE),
                      global_feats, wb1_bf, bb1.reshape(1, H), wb2_bf,
                      bb2.reshape(1, D), 0)
    e_new2 = _tc_edge(hsum2, bond, gsrc2.reshape(EH // BE, 1, BE),
                      global_feats, wb1_bf, bb1.reshape(1, H), wb2_bf,
                      bb2.reshape(1, D), EH // BE)
    sume1_parts = _sc_scatter(e_new1, src, dst, zeros_nd, 0, EH,
                              gather_rows=False)
    sume2_parts = _sc_scatter(e_new2, src, dst, zeros_nd, EH, EH,
                              gather_rows=False)
    buf = jnp.zeros((E, D), jnp.float32)
    buf = lax.dynamic_update_slice(buf, e_new1, (0, 0))
    e_new = lax.dynamic_update_slice(buf, e_new2, (EH, 0))
    n2g3 = n2g.reshape(N // BN, 1, BN)
    h_new, u_new = _tc_final(atom, sumh_parts, sume1_parts, sume2_parts,
                             n2g3, global_feats, Wa1, ba1.reshape(1, H),
                             Wa2, ba2.reshape(1, D), Wg1, bg1.reshape(1, H),
                             Wg2, bg2.reshape(1, D))
    return (h_new, e_new, u_new)


# speedup vs baseline: 1.0380x; 1.0380x over previous
import jax
import jax.numpy as jnp
from jax import lax
from jax.experimental import pallas as pl
from jax.experimental.pallas import tpu as pltpu
from jax.experimental.pallas import tpu_sc as plsc

N = 10000
E = 320000
G = 256
D = 128
H = 3 * D

NSPLIT = 2
EH = E // NSPLIT

NC = 2
NS = 16
NW = NC * NS

RPT = 624
RTAIL = N - NS * RPT

BE = 1280
BN = 1000


def _sc_mesh():
    return plsc.VectorSubcoreMesh(core_axis_name="c", subcore_axis_name="s")


def _pick_chunk(per_worker, limit):
    for c in (200, 160, 80, 40, 16, 8):
        if c <= limit and per_worker % c == 0 and (per_worker // c) % 2 == 1:
            return c
    raise ValueError(per_worker)


def _sc_gather(atom, n2g, src, dst, base, esize):
    ew = esize // NW
    cg = _pick_chunk(ew, 80)
    nchunk = ew // cg

    def body(atom_hbm, n2g_hbm, src_hbm, dst_hbm, hsum_hbm, gsrc_hbm,
             idx_s, idx_d, n2g_v, g_all, rs0, rd0, rs1, rd1, hs0, hs1,
             semg0, semg1, semw0, semw1):
        cid = lax.axis_index("c")
        sid = lax.axis_index("s")
        wid = sid * NC + cid
        gbase = base + wid * ew
        obase = wid * ew

        pltpu.sync_copy(src_hbm.at[pl.ds(gbase, ew)], idx_s)
        pltpu.sync_copy(dst_hbm.at[pl.ds(gbase, ew)], idx_d)
        pltpu.sync_copy(n2g_hbm, n2g_v)

        def fire(k, rs, rd, sem):
            pltpu.async_copy(atom_hbm.at[idx_s.at[pl.ds(k * cg, cg)]], rs, sem)
            pltpu.async_copy(atom_hbm.at[idx_d.at[pl.ds(k * cg, cg)]], rd, sem)

        def drain_g(rs, rd, sem):
            pltpu.make_async_copy(atom_hbm.at[pl.ds(0, cg)], rs, sem).wait()
            pltpu.make_async_copy(atom_hbm.at[pl.ds(0, cg)], rd, sem).wait()

        def drain_w(hs, sem):
            pltpu.make_async_copy(hs, hsum_hbm.at[pl.ds(0, cg)], sem).wait()

        def process(k, rs, rd, hs, semw):
            def row(r, c):
                def col(q, c2):
                    sl = pl.ds(q * 16, 16)
                    hs[r, sl] = rs[r, sl] + rd[r, sl]
                    return c2
                return lax.fori_loop(0, D // 16, col, c)
            lax.fori_loop(0, cg, row, 0)
            pltpu.async_copy(hs, hsum_hbm.at[pl.ds(obase + k * cg, cg)], semw)

        fire(0, rs0, rd0, semg0)

        def gs(j, c):
            sl = pl.ds(j * 16, 16)
            g_all[sl] = plsc.load_gather(n2g_v, [idx_s[sl]])
            return c
        lax.fori_loop(0, ew // 16, gs, 0)
        if ew % 16:
            sl = pl.ds(ew - 16, 16)
            g_all[sl] = plsc.load_gather(n2g_v, [idx_s[sl]])
        pltpu.sync_copy(g_all, gsrc_hbm.at[pl.ds(obase, ew)])

        def pair(t, carry):
            a = 2 * t
            b = a + 1
            fire(b, rs1, rd1, semg1)
            drain_g(rs0, rd0, semg0)

            @pl.when(t > 0)
            def _():
                drain_w(hs0, semw0)
            process(a, rs0, rd0, hs0, semw0)
            fire(b + 1, rs0, rd0, semg0)
            drain_g(rs1, rd1, semg1)

            @pl.when(t > 0)
            def _():
                drain_w(hs1, semw1)
            process(b, rs1, rd1, hs1, semw1)
            return carry

        lax.fori_loop(0, (nchunk - 1) // 2, pair, 0)
        drain_g(rs0, rd0, semg0)
        drain_w(hs0, semw0)
        process(nchunk - 1, rs0, rd0, hs0, semw0)
        drain_w(hs0, semw0)
        drain_w(hs1, semw1)

    f = pl.kernel(
        body,
        out_type=(
            jax.ShapeDtypeStruct((esize, D), jnp.float32),
            jax.ShapeDtypeStruct((esize,), jnp.int32),
        ),
        mesh=_sc_mesh(),
        scratch_types=[
            pltpu.VMEM((ew,), jnp.int32),
            pltpu.VMEM((ew,), jnp.int32),
            pltpu.VMEM((N,), jnp.int32),
            pltpu.VMEM((ew,), jnp.int32),
            pltpu.VMEM((cg, D), jnp.float32),
            pltpu.VMEM((cg, D), jnp.float32),
            pltpu.VMEM((cg, D), jnp.float32),
            pltpu.VMEM((cg, D), jnp.float32),
            pltpu.VMEM((cg, D), jnp.float32),
            pltpu.VMEM((cg, D), jnp.float32),
            pltpu.SemaphoreType.DMA,
            pltpu.SemaphoreType.DMA,
            pltpu.SemaphoreType.DMA,
            pltpu.SemaphoreType.DMA,
        ],
        compiler_params=pltpu.CompilerParams(needs_layout_passes=False),
    )
    return f(atom, n2g, src, dst)


def _sc_scatter(data, src, dst, zeros_nd, base_idx, esize, gather_rows):
    ewl = esize // NW
    cr = 40
    nchunk = ewl // cr

    def body(data_hbm, src_hbm, dst_hbm, zeros_hbm, out_hbm,
             idx_s, i0, i1, i2, i3, r0, r1, r2, r3, accum,
             semx0, semx1, semx2, semx3,
             semr0, semr1, semr2, semr3,
             semc0, semc1, semc2, semc3):
        idd = (i0, i1, i2, i3)
        rows = (r0, r1, r2, r3)
        semx = (semx0, semx1, semx2, semx3)
        semr = (semr0, semr1, semr2, semr3)
        semc = (semc0, semc1, semc2, semc3)
        cid = lax.axis_index("c")
        sid = lax.axis_index("s")
        wid = sid * NC + cid
        ibase = base_idx + wid * ewl
        dbase = wid * ewl

        rr0 = sid * RPT
        pltpu.sync_copy(zeros_hbm.at[pl.ds(rr0, RPT)],
                        accum.at[pl.ds(rr0, RPT)])

        @pl.when(sid == 0)
        def _():
            pltpu.sync_copy(zeros_hbm.at[pl.ds(NS * RPT, RTAIL)],
                            accum.at[pl.ds(NS * RPT, RTAIL)])
        if gather_rows:
            pltpu.sync_copy(src_hbm.at[pl.ds(ibase, ewl)], idx_s)
        plsc.subcore_barrier()

        def fire_idx(k, slot):
            pltpu.async_copy(dst_hbm.at[pl.ds(ibase + k * cr, cr)],
                             idd[slot], semx[slot])

        def drain_idx(slot):
            pltpu.make_async_copy(dst_hbm.at[pl.ds(0, cr)], idd[slot],
                                  semx[slot]).wait()

        def fire_rows(k, slot):
            if gather_rows:
                pltpu.async_copy(data_hbm.at[idx_s.at[pl.ds(k * cr, cr)]],
                                 rows[slot], semr[slot])
            else:
                pltpu.async_copy(data_hbm.at[pl.ds(dbase + k * cr, cr)],
                                 rows[slot], semr[slot])

        def drain_rows(slot):
            pltpu.make_async_copy(data_hbm.at[pl.ds(0, cr)], rows[slot],
                                  semr[slot]).wait()

        def fire_sc(k, slot):
            pltpu.async_copy(rows[slot], accum.at[idd[slot]], semc[slot],
                             add=True)

        def drain_sc(slot):
            pltpu.make_async_copy(rows[slot], accum.at[pl.ds(0, cr)],
                                  semc[slot]).wait()

        for j in range(3):
            fire_idx(j, j)
            fire_rows(j, j)

        def quad(t, c):
            for j in range(4):
                k = 4 * t + j

                @pl.when(k < nchunk)
                def _():
                    drain_rows(j)
                    drain_idx(j)
                    fire_sc(k, j)
                    kk = k + 3

                    @pl.when(kk < nchunk)
                    def _():
                        jj = (j + 3) % 4

                        @pl.when(kk >= 4)
                        def _():
                            drain_sc(jj)
                        fire_idx(kk, jj)
                        fire_rows(kk, jj)
            return c

        lax.fori_loop(0, (nchunk + 3) // 4, quad, 0)
        for j in range(4):
            drain_sc(j)

        plsc.subcore_barrier()
        pltpu.sync_copy(accum.at[pl.ds(rr0, RPT)],
                        out_hbm.at[cid, pl.ds(rr0, RPT)])

        @pl.when(sid == 0)
        def _():
            pltpu.sync_copy(accum.at[pl.ds(NS * RPT, RTAIL)],
                            out_hbm.at[cid, pl.ds(NS * RPT, RTAIL)])

    f = pl.kernel(
        body,
        out_type=jax.ShapeDtypeStruct((NC, N, D), jnp.float32),
        mesh=_sc_mesh(),
        scratch_types=(
            [pltpu.VMEM((ewl,), jnp.int32)]
            + [pltpu.VMEM((cr,), jnp.int32) for _ in range(4)]
            + [pltpu.VMEM((cr, D), jnp.float32) for _ in range(4)]
            + [pltpu.VMEM_SHARED((N, D), jnp.float32)]
            + [pltpu.SemaphoreType.DMA for _ in range(12)]
        ),
    )
    return f(data, src, dst, zeros_nd)


def _tc_edge_body(hsum_ref, bond_ref, gsrc_ref, u_ref, wb1_ref,
                  bb1_ref, wb2_ref, bb2_ref, out_ref, ugb_ref):
    i = pl.program_id(0)

    @pl.when(i == 0)
    def _():
        ugb_ref[...] = (
            jnp.dot(u_ref[...].astype(jnp.bfloat16), wb1_ref[2 * D:, :],
                    preferred_element_type=jnp.float32)
            + bb1_ref[...]).astype(jnp.bfloat16)

    g = gsrc_ref[0]
    ohT = (lax.broadcasted_iota(jnp.int32, (G, BE), 0) == g
           ).astype(jnp.bfloat16)
    xu = lax.dot_general(ohT, ugb_ref[...], (((0,), (0,)), ((), ())),
                         preferred_element_type=jnp.float32)
    sh = hsum_ref[...].astype(jnp.bfloat16)
    shb = jnp.concatenate([sh, bond_ref[...].astype(jnp.bfloat16)], axis=1)
    x = (jnp.dot(shb, wb1_ref[:2 * D, :], preferred_element_type=jnp.float32)
         + xu)
    x = jnp.maximum(x, 0.0).astype(jnp.bfloat16)
    out_ref[...] = (jnp.dot(x, wb2_ref[...],
                            preferred_element_type=jnp.float32)
                    + bb2_ref[...])


def _tc_edge(hsum, bond, gsrc3, u, Wb1, bb1, Wb2, bb2, boff):
    nsteps = EH // BE
    return pl.pallas_call(
        _tc_edge_body,
        grid=(nsteps,),
        in_specs=[
            pl.BlockSpec((BE, D), lambda i: (i, 0)),
            pl.BlockSpec((BE, D), lambda i: (i + boff, 0)),
            pl.BlockSpec((1, 1, BE), lambda i: (i, 0, 0)),
            pl.BlockSpec((G, D), lambda i: (0, 0)),
            pl.BlockSpec((H, H), lambda i: (0, 0)),
            pl.BlockSpec((1, H), lambda i: (0, 0)),
            pl.BlockSpec((H, D), lambda i: (0, 0)),
            pl.BlockSpec((1, D), lambda i: (0, 0)),
        ],
        out_specs=pl.BlockSpec((BE, D), lambda i: (i, 0)),
        out_shape=jax.ShapeDtypeStruct((EH, D), jnp.float32),
        scratch_shapes=[pltpu.VMEM((G, H), jnp.bfloat16)],
        compiler_params=pltpu.CompilerParams(
            dimension_semantics=("arbitrary",)),
    )(hsum, bond, gsrc3, u, Wb1, bb1, Wb2, bb2)


def _tc_final_body(atom_ref, sumh_ref, sume1_ref, sume2_ref, n2g_ref, u_ref,
                   wa1_ref, ba1_ref, wa2_ref, ba2_ref,
                   wg1_ref, bg1_ref, wg2_ref, bg2_ref,
                   hnew_ref, unew_ref,
                   segh_ref, sege_ref, uga_ref):
    i = pl.program_id(0)
    nsteps = pl.num_programs(0)

    @pl.when(i == 0)
    def _():
        segh_ref[...] = jnp.zeros((G, D), jnp.float32)
        sege_ref[...] = jnp.zeros((G, D), jnp.float32)
        uga_ref[...] = (
            jnp.dot(u_ref[...], wa1_ref[2 * D:, :],
                    preferred_element_type=jnp.float32) + ba1_ref[...])

    sum_h = sumh_ref[0] + sumh_ref[1] + atom_ref[...]
    sum_e = (sume1_ref[0] + sume1_ref[1]) + (sume2_ref[0] + sume2_ref[1])
    g = n2g_ref[0]
    ohT = (lax.broadcasted_iota(jnp.int32, (G, BN), 0) == g
           ).astype(jnp.float32)
    xu = lax.dot_general(ohT, uga_ref[...], (((0,), (0,)), ((), ())),
                         preferred_element_type=jnp.float32)
    x = (jnp.dot(sum_h, wa1_ref[:D, :], preferred_element_type=jnp.float32)
         + jnp.dot(sum_e, wa1_ref[D:2 * D, :],
                   preferred_element_type=jnp.float32)
         + xu)
    x = jnp.maximum(x, 0.0)
    hn = (jnp.dot(x, wa2_ref[...], preferred_element_type=jnp.float32)
          + ba2_ref[...])
    hnew_ref[...] = hn
    segh_ref[...] += jnp.dot(ohT, hn, preferred_element_type=jnp.float32)
    sege_ref[...] += jnp.dot(ohT, sum_e, preferred_element_type=jnp.float32)

    @pl.when(i == nsteps - 1)
    def _():
        xg = (jnp.dot(segh_ref[...], wg1_ref[:D, :],
                      preferred_element_type=jnp.float32)
              + jnp.dot(0.5 * sege_ref[...], wg1_ref[D:2 * D, :],
                        preferred_element_type=jnp.float32)
              + jnp.dot(u_ref[...], wg1_ref[2 * D:, :],
                        preferred_element_type=jnp.float32)
              + bg1_ref[...])
        xg = jnp.maximum(xg, 0.0)
        unew_ref[...] = (jnp.dot(xg, wg2_ref[...],
                                 preferred_element_type=jnp.float32)
                         + bg2_ref[...])


def _tc_final(atom, sumh_parts, sume1_parts, sume2_parts, n2g3, u,
              Wa1, ba1, Wa2, ba2, Wg1, bg1, Wg2, bg2):
    nsteps = N // BN
    return pl.pallas_call(
        _tc_final_body,
        grid=(nsteps,),
        in_specs=[
            pl.BlockSpec((BN, D), lambda i: (i, 0)),
            pl.BlockSpec((NC, BN, D), lambda i: (0, i, 0)),
            pl.BlockSpec((NC, BN, D), lambda i: (0, i, 0)),
            pl.BlockSpec((NC, BN, D), lambda i: (0, i, 0)),
            pl.BlockSpec((1, 1, BN), lambda i: (i, 0, 0)),
            pl.BlockSpec((G, D), lambda i: (0, 0)),
            pl.BlockSpec((H, H), lambda i: (0, 0)),
            pl.BlockSpec((1, H), lambda i: (0, 0)),
            pl.BlockSpec((H, D), lambda i: (0, 0)),
            pl.BlockSpec((1, D), lambda i: (0, 0)),
            pl.BlockSpec((H, H), lambda i: (0, 0)),
            pl.BlockSpec((1, H), lambda i: (0, 0)),
            pl.BlockSpec((H, D), lambda i: (0, 0)),
            pl.BlockSpec((1, D), lambda i: (0, 0)),
        ],
        out_specs=[
            pl.BlockSpec((BN, D), lambda i: (i, 0)),
            pl.BlockSpec((G, D), lambda i: (0, 0)),
        ],
        out_shape=[
            jax.ShapeDtypeStruct((N, D), jnp.float32),
            jax.ShapeDtypeStruct((G, D), jnp.float32),
        ],
        scratch_shapes=[
            pltpu.VMEM((G, D), jnp.float32),
            pltpu.VMEM((G, D), jnp.float32),
            pltpu.VMEM((G, H), jnp.float32),
        ],
        compiler_params=pltpu.CompilerParams(
            dimension_semantics=("arbitrary",)),
    )(atom, sumh_parts, sume1_parts, sume2_parts, n2g3, u,
      Wa1, ba1, Wa2, ba2, Wg1, bg1, Wg2, bg2)


def kernel(atom, bond, global_feats, edge_index, node2graph,
           Wb1, bb1, Wb2, bb2, Wa1, ba1, Wa2, ba2, Wg1, bg1, Wg2, bg2):
    src = edge_index[0].astype(jnp.int32)
    dst = edge_index[1].astype(jnp.int32)
    n2g = node2graph.astype(jnp.int32)
    zeros_nd = jnp.zeros((N, D), jnp.float32)
    wb1_bf = Wb1.astype(jnp.bfloat16)
    wb2_bf = Wb2.astype(jnp.bfloat16)

    hsum1, gsrc1 = _sc_gather(atom, n2g, src, dst, 0, EH)
    hsum2, gsrc2 = _sc_gather(atom, n2g, src, dst, EH, EH)
    sumh_parts = _sc_scatter(atom, src, dst, zeros_nd, 0, E,
                             gather_rows=True)
    e_new1 = _tc_edge(hsum1, bond, gsrc1.reshape(EH // BE, 1, BE),
                      global_feats, wb1_bf, bb1.reshape(1, H), wb2_bf,
                      bb2.reshape(1, D), 0)
    e_new2 = _tc_edge(hsum2, bond, gsrc2.reshape(EH // BE, 1, BE),
                      global_feats, wb1_bf, bb1.reshape(1, H), wb2_bf,
                      bb2.reshape(1, D), EH // BE)
    sume1_parts = _sc_scatter(e_new1, src, dst, zeros_nd, 0, EH,
                              gather_rows=False)
    sume2_parts = _sc_scatter(e_new2, src, dst, zeros_nd, EH, EH,
                              gather_rows=False)
    e_new = jnp.concatenate([e_new1, e_new2], axis=0)
    n2g3 = n2g.reshape(N // BN, 1, BN)
    h_new, u_new = _tc_final(atom, sumh_parts, sume1_parts, sume2_parts,
                             n2g3, global_feats, Wa1, ba1.reshape(1, H),
                             Wa2, ba2.reshape(1, D), Wg1, bg1.reshape(1, H),
                             Wg2, bg2.reshape(1, D))
    return (h_new, e_new, u_new)
